# baseline (speedup 1.0000x reference)
import functools

import jax
import jax.numpy as jnp
from jax import lax
from jax.experimental import pallas as pl
from jax.experimental.pallas import tpu as pltpu
from jax.experimental.pallas import tpu_sc as plsc

EMBED = 64
TC_BLK = 1000
ROW_BLK = 32


def _normalize_body(m_ref, out_ref):
    x = m_ref[...]
    s = jnp.sum(x * x, axis=1, keepdims=True)
    y = x / (jnp.sqrt(s) + 1e-8)
    out_ref[...] = jnp.concatenate([y, y], axis=1)


def _normalize_table(matrix):
    vocab = matrix.shape[0]
    grid = vocab // TC_BLK
    return pl.pallas_call(
        _normalize_body,
        grid=(grid,),
        in_specs=[pl.BlockSpec((TC_BLK, EMBED), lambda i: (i, 0))],
        out_specs=pl.BlockSpec((TC_BLK, 2 * EMBED), lambda i: (i, 0)),
        out_shape=jax.ShapeDtypeStruct((vocab, 2 * EMBED), jnp.float32),
        compiler_params=pltpu.CompilerParams(
            dimension_semantics=("arbitrary",)),
    )(matrix)


def _narrow_body(g_ref, out_ref):
    out_ref[...] = g_ref[:, :, :EMBED]


def _narrow_rows(gathered):
    nb, nt, _ = gathered.shape
    grid = nb // ROW_BLK
    return pl.pallas_call(
        _narrow_body,
        grid=(grid,),
        in_specs=[pl.BlockSpec((ROW_BLK, nt, 2 * EMBED),
                               lambda i: (i, 0, 0))],
        out_specs=pl.BlockSpec((ROW_BLK, nt, EMBED), lambda i: (i, 0, 0)),
        out_shape=jax.ShapeDtypeStruct((nb, nt, EMBED), jnp.float32),
        compiler_params=pltpu.CompilerParams(
            dimension_semantics=("arbitrary",)),
    )(gathered)


@functools.cache
def _build_gather(nb, nt):
    info = plsc.get_sparse_core_info()
    nc, ns = info.num_cores, info.num_subcores
    nw = nc * ns
    rows_w = nb // nw
    n1 = 128
    n2 = nt - n1

    mesh = plsc.VectorSubcoreMesh(core_axis_name="c", subcore_axis_name="s")

    @functools.partial(
        pl.kernel,
        mesh=mesh,
        compiler_params=pltpu.CompilerParams(
            needs_layout_passes=False, use_tc_tiling_on_sc=True),
        out_type=jax.ShapeDtypeStruct((nb, nt, 2 * EMBED), jnp.float32),
        scratch_types=[
            pltpu.VMEM((nt,), jnp.int32),
            pltpu.VMEM((nt, 2 * EMBED), jnp.float32),
            pltpu.SemaphoreType.DMA,
        ],
    )
    def sc_fn(m2_hbm, tok_hbm, out_hbm, idx_v, rows_v, sem):
        wid = lax.axis_index("s") * nc + lax.axis_index("c")

        def row_body(t, carry):
            b = wid * rows_w + t
            pltpu.sync_copy(tok_hbm.at[b], idx_v)
            h1 = pltpu.async_copy(
                m2_hbm.at[idx_v.at[pl.ds(0, n1)]],
                rows_v.at[pl.ds(0, n1)], sem)
            h2 = pltpu.async_copy(
                m2_hbm.at[idx_v.at[pl.ds(n1, n2)]],
                rows_v.at[pl.ds(n1, n2)], sem)
            h1.wait()
            h2.wait()
            pltpu.sync_copy(rows_v, out_hbm.at[b])
            return carry

        lax.fori_loop(0, rows_w, row_body, 0)

    return sc_fn


def kernel(matrix, tokens):
    nb, nt = tokens.shape
    tok = tokens.astype(jnp.int32)
    m2 = _normalize_table(matrix)
    gathered = _build_gather(nb, nt)(m2, tok)
    return _narrow_rows(gathered)


# speedup vs baseline: 1.1511x; 1.1511x over previous
import functools

import jax
import jax.numpy as jnp
from jax import lax
from jax.experimental import pallas as pl
from jax.experimental.pallas import tpu as pltpu
from jax.experimental.pallas import tpu_sc as plsc

EMBED = 64
ROW_BLK = 32


def _select_norm_body(g_ref, tok_ref, out_ref):
    g = g_ref[...]
    par = (tok_ref[...] & 1)[:, :, None]
    x = jnp.where(par == 1, g[:, :, EMBED:], g[:, :, :EMBED])
    s = jnp.sum(x * x, axis=2, keepdims=True)
    out_ref[...] = x / (jnp.sqrt(s) + 1e-8)


def _select_norm(gathered, tokens):
    nb, nt, _ = gathered.shape
    grid = nb // ROW_BLK
    return pl.pallas_call(
        _select_norm_body,
        grid=(grid,),
        in_specs=[
            pl.BlockSpec((ROW_BLK, nt, 2 * EMBED), lambda i: (i, 0, 0)),
            pl.BlockSpec((ROW_BLK, nt), lambda i: (i, 0)),
        ],
        out_specs=pl.BlockSpec((ROW_BLK, nt, EMBED), lambda i: (i, 0, 0)),
        out_shape=jax.ShapeDtypeStruct((nb, nt, EMBED), jnp.float32),
        compiler_params=pltpu.CompilerParams(
            dimension_semantics=("arbitrary",)),
    )(gathered, tokens)


@functools.cache
def _build_gather(nb, nt):
    info = plsc.get_sparse_core_info()
    nc, ns = info.num_cores, info.num_subcores
    nw = nc * ns
    rows_w = nb // nw
    n1 = 128
    n2 = nt - n1

    mesh = plsc.VectorSubcoreMesh(core_axis_name="c", subcore_axis_name="s")

    @functools.partial(
        pl.kernel,
        mesh=mesh,
        compiler_params=pltpu.CompilerParams(
            needs_layout_passes=False, use_tc_tiling_on_sc=True),
        out_type=jax.ShapeDtypeStruct((nb, nt, 2 * EMBED), jnp.float32),
        scratch_types=[
            pltpu.VMEM((nt,), jnp.int32),
            pltpu.VMEM((nt, 2 * EMBED), jnp.float32),
            pltpu.SemaphoreType.DMA,
        ],
    )
    def sc_fn(m2_hbm, tok_hbm, out_hbm, idx_v, rows_v, sem):
        wid = lax.axis_index("s") * nc + lax.axis_index("c")

        def row_body(t, carry):
            b = wid * rows_w + t
            pltpu.sync_copy(tok_hbm.at[b], idx_v)
            h1 = pltpu.async_copy(
                m2_hbm.at[idx_v.at[pl.ds(0, n1)]],
                rows_v.at[pl.ds(0, n1)], sem)
            h2 = pltpu.async_copy(
                m2_hbm.at[idx_v.at[pl.ds(n1, n2)]],
                rows_v.at[pl.ds(n1, n2)], sem)
            h1.wait()
            h2.wait()
            pltpu.sync_copy(rows_v, out_hbm.at[b])
            return carry

        lax.fori_loop(0, rows_w, row_body, 0)

    return sc_fn


def kernel(matrix, tokens):
    nb, nt = tokens.shape
    vocab = matrix.shape[0]
    tok = tokens.astype(jnp.int32)
    packed = matrix.reshape(vocab // 2, 2 * EMBED)
    gathered = _build_gather(nb, nt)(packed, tok >> 1)
    return _select_norm(gathered, tok)


# speedup vs baseline: 1.4192x; 1.2329x over previous
import functools

import jax
import jax.numpy as jnp
from jax import lax
from jax.experimental import pallas as pl
from jax.experimental.pallas import tpu as pltpu
from jax.experimental.pallas import tpu_sc as plsc

EMBED = 64
ROW_BLK = 32


def _select_norm_body(g_ref, tok_ref, out_ref):
    g = g_ref[...]
    i = lax.broadcasted_iota(jnp.int32, (2 * EMBED, 2 * EMBED), 0)
    j = lax.broadcasted_iota(jnp.int32, (2 * EMBED, 2 * EMBED), 1)
    blk = jnp.where((i < EMBED) == (j < EMBED), 1.0, 0.0)
    g2 = g * g
    s = jnp.stack(
        [lax.dot_general(g2[k], blk, (((1,), (0,)), ((), ())),
                         preferred_element_type=jnp.float32)
         for k in range(g.shape[0])], axis=0)
    y = g / (jnp.sqrt(s) + 1e-8)
    par = (tok_ref[...] & 1)[:, :, None]
    rot = jnp.concatenate([y[:, :, EMBED:], y[:, :, :EMBED]], axis=2)
    out_ref[...] = jnp.where(par == 1, rot, y)[:, :, :EMBED]


def _select_norm(gathered, tokens):
    nb, nt, _ = gathered.shape
    grid = nb // ROW_BLK
    return pl.pallas_call(
        _select_norm_body,
        grid=(grid,),
        in_specs=[
            pl.BlockSpec((ROW_BLK, nt, 2 * EMBED), lambda i: (i, 0, 0)),
            pl.BlockSpec((ROW_BLK, nt), lambda i: (i, 0)),
        ],
        out_specs=pl.BlockSpec((ROW_BLK, nt, EMBED), lambda i: (i, 0, 0)),
        out_shape=jax.ShapeDtypeStruct((nb, nt, EMBED), jnp.float32),
        compiler_params=pltpu.CompilerParams(
            dimension_semantics=("arbitrary",)),
    )(gathered, tokens)


@functools.cache
def _build_gather(nb, nt):
    info = plsc.get_sparse_core_info()
    nc, ns = info.num_cores, info.num_subcores
    nw = nc * ns
    rows_w = nb // nw
    n1 = 128
    n2 = nt - n1

    mesh = plsc.VectorSubcoreMesh(core_axis_name="c", subcore_axis_name="s")

    @functools.partial(
        pl.kernel,
        mesh=mesh,
        compiler_params=pltpu.CompilerParams(
            needs_layout_passes=False, use_tc_tiling_on_sc=True),
        out_type=jax.ShapeDtypeStruct((nb, nt, 2 * EMBED), jnp.float32),
        scratch_types=[
            pltpu.VMEM((2, nt), jnp.int32),
            pltpu.VMEM((2, nt, 2 * EMBED), jnp.float32),
            pltpu.SemaphoreType.DMA,
            pltpu.SemaphoreType.DMA,
            pltpu.SemaphoreType.DMA,
        ],
    )
    def sc_fn(m2_hbm, tok_hbm, out_hbm, idx2_v, rows2_v, sem_i, sem_g,
              sem_w):
        wid = lax.axis_index("s") * nc + lax.axis_index("c")
        base = wid * rows_w

        pltpu.async_copy(tok_hbm.at[base], idx2_v.at[0], sem_i)

        def row_body(t, carry):
            p = t & 1
            pltpu.make_async_copy(
                tok_hbm.at[base], idx2_v.at[p], sem_i).wait()
            tn = jnp.minimum(t + 1, rows_w - 1)
            pltpu.async_copy(
                tok_hbm.at[base + tn], idx2_v.at[1 - p], sem_i)
            @pl.when(t >= 2)
            def _():
                pltpu.make_async_copy(
                    rows2_v.at[p], out_hbm.at[base], sem_w).wait()
            h1 = pltpu.async_copy(
                m2_hbm.at[idx2_v.at[p, pl.ds(0, n1)]],
                rows2_v.at[p, pl.ds(0, n1)], sem_g)
            h2 = pltpu.async_copy(
                m2_hbm.at[idx2_v.at[p, pl.ds(n1, n2)]],
                rows2_v.at[p, pl.ds(n1, n2)], sem_g)
            h1.wait()
            h2.wait()
            pltpu.async_copy(rows2_v.at[p], out_hbm.at[base + t], sem_w)
            return carry

        lax.fori_loop(0, rows_w, row_body, 0)

        pltpu.make_async_copy(rows2_v.at[0], out_hbm.at[base], sem_w).wait()
        pltpu.make_async_copy(rows2_v.at[1], out_hbm.at[base], sem_w).wait()
        pltpu.make_async_copy(
            tok_hbm.at[base], idx2_v.at[0], sem_i).wait()

    return sc_fn


def kernel(matrix, tokens):
    nb, nt = tokens.shape
    vocab = matrix.shape[0]
    tok = tokens.astype(jnp.int32)
    packed = matrix.reshape(vocab // 2, 2 * EMBED)
    gathered = _build_gather(nb, nt)(packed, tok >> 1)
    return _select_norm(gathered, tok)
